# baseline (speedup 1.0000x reference)
import functools

import jax
import jax.numpy as jnp
from jax import lax
from jax.experimental import pallas as pl
from jax.experimental.pallas import tpu as pltpu
from jax.experimental.pallas import tpu_sc as plsc

_NC = 2
_NS = 16
_NW = _NC * _NS
_N = 16384
_K = 3
_D = 16
_J = _K * _D
_V = 1024
_NPW = _N // _NW
_L = 16
_GRP = _NPW // _L
_XP = 9
_XR = 20


def _build_gather():
    mesh = plsc.VectorSubcoreMesh(core_axis_name="c", subcore_axis_name="s")

    @functools.partial(
        pl.kernel,
        mesh=mesh,
        out_type=(
            jax.ShapeDtypeStruct((_J, _N), jnp.float32),
            jax.ShapeDtypeStruct((_XP, _XR, _N), jnp.float32),
        ),
        scratch_types=[
            pltpu.VMEM((_K * _NPW,), jnp.int32),
            pltpu.VMEM((_V * _D,), jnp.float32),
            pltpu.VMEM((_J * _NPW,), jnp.float32),
            pltpu.VMEM((4, _XR, _NPW), jnp.float32),
            pltpu.SemaphoreType.DMA,
            pltpu.SemaphoreType.DMA,
            pltpu.SemaphoreType.DMA,
        ],
        compiler_params=pltpu.CompilerParams(needs_layout_passes=False),
    )
    def gather_kernel(kmer_t_hbm, table_hbm, x_hbm, out_hbm, xout_hbm,
                      idx_v, tab_v, m_v, x_v, sem, xrsem, xwsem):
        wid = lax.axis_index("s") * _NC + lax.axis_index("c")
        n0 = wid * _NPW

        def x_read(p):
            return pltpu.async_copy(
                x_hbm.at[p, :, pl.ds(n0, _NPW)], x_v.at[p % 4], xrsem)

        def x_write(p):
            return pltpu.async_copy(
                x_v.at[p % 4], xout_hbm.at[p, :, pl.ds(n0, _NPW)], xwsem)

        xr = {p: x_read(p) for p in range(4)}
        xw = {}
        pltpu.sync_copy(table_hbm, tab_v)
        for k in range(_K):
            pltpu.sync_copy(
                kmer_t_hbm.at[pl.ds(k * _N + n0, _NPW)],
                idx_v.at[pl.ds(k * _NPW, _NPW)],
            )

        m_copies = []
        for s in range(_K):

            @plsc.parallel_loop(0, _GRP, unroll=8)
            def _body(i):
                base = i * _L
                voff = idx_v[pl.ds(s * _NPW + base, _L)] * _D
                vals = [plsc.load_gather(tab_v, [voff + c]) for c in range(_D)]
                for c in range(_D):
                    m_v[pl.ds((s * _D + c) * _NPW + base, _L)] = vals[c]

            for c in range(_D):
                j = s * _D + c
                m_copies.append(
                    pltpu.async_copy(
                        m_v.at[pl.ds(j * _NPW, _NPW)],
                        out_hbm.at[j, pl.ds(n0, _NPW)],
                        sem,
                    )
                )
            if s == 0:
                for p in range(3):
                    xr[p].wait()
                    xw[p] = x_write(p)
            elif s == 1:
                for p in range(3):
                    xw[p].wait()
                    xr[p + 4] = x_read(p + 4)
                xr[3].wait()
                xw[3] = x_write(3)
            else:
                xw[3].wait()
                xr[7] = x_read(7)
                for p in range(4, 6):
                    xr[p].wait()
                    xw[p] = x_write(p)

        xw[4].wait()
        xr[8] = x_read(8)
        for p in (6, 7, 8):
            xr[p].wait()
            xw[p] = x_write(p)
        for h in m_copies:
            h.wait()
        for p in (5, 6, 7, 8):
            xw[p].wait()

    return gather_kernel


_gather = _build_gather()


def kernel(X, kmer, emb_table):
    kmer_t = kmer.astype(jnp.int32).T.reshape(-1)
    m, xo = _gather(kmer_t, emb_table.reshape(-1), X.transpose(2, 1, 0))
    return (xo.transpose(2, 1, 0), m.T)


# speedup vs baseline: 1.2747x; 1.2747x over previous
import functools

import jax
import jax.numpy as jnp
from jax import lax
from jax.experimental import pallas as pl
from jax.experimental.pallas import tpu as pltpu
from jax.experimental.pallas import tpu_sc as plsc

_NC = 2
_NS = 16
_NW = _NC * _NS
_N = 16384
_K = 3
_D = 16
_J = _K * _D
_V = 1024
_NPW = _N // _NW
_L = 16
_GRP = _NPW // _L
_XP = 9
_XR = 20


def _build_gather():
    mesh = plsc.VectorSubcoreMesh(core_axis_name="c", subcore_axis_name="s")

    @functools.partial(
        pl.kernel,
        mesh=mesh,
        out_type=(
            jax.ShapeDtypeStruct((_J, _N), jnp.float32),
            jax.ShapeDtypeStruct((_XP, _XR, _N), jnp.float32),
        ),
        scratch_types=[
            pltpu.VMEM((_K * _NPW,), jnp.int32),
            pltpu.VMEM((_V * _D,), jnp.float32),
            pltpu.VMEM((_J * _NPW,), jnp.float32),
            pltpu.VMEM((4, _XR, _NPW), jnp.float32),
            pltpu.SemaphoreType.DMA,
            pltpu.SemaphoreType.DMA,
            pltpu.SemaphoreType.DMA,
        ],
        compiler_params=pltpu.CompilerParams(needs_layout_passes=False),
    )
    def gather_kernel(kmer_t_hbm, table_hbm, x_hbm, out_hbm, xout_hbm,
                      idx_v, tab_v, m_v, x_v, sem, xrsem, xwsem):
        wid = lax.axis_index("s") * _NC + lax.axis_index("c")
        n0 = wid * _NPW

        def x_read(p):
            return pltpu.async_copy(
                x_hbm.at[p, :, pl.ds(n0, _NPW)], x_v.at[p % 4], xrsem)

        def x_write(p):
            return pltpu.async_copy(
                x_v.at[p % 4], xout_hbm.at[p, :, pl.ds(n0, _NPW)], xwsem)

        xr = {p: x_read(p) for p in range(4)}
        xw = {}
        pltpu.sync_copy(table_hbm, tab_v)
        for k in range(_K):
            pltpu.sync_copy(
                kmer_t_hbm.at[pl.ds(k * _N + n0, _NPW)],
                idx_v.at[pl.ds(k * _NPW, _NPW)],
            )

        m_copies = []
        for s in range(_K):

            @plsc.parallel_loop(0, _GRP, unroll=2)
            def _body(i):
                base = i * _L
                voff = idx_v[pl.ds(s * _NPW + base, _L)] * _D
                vals = [plsc.load_gather(tab_v, [voff + c]) for c in range(_D)]
                for c in range(_D):
                    m_v[pl.ds((s * _D + c) * _NPW + base, _L)] = vals[c]

            for c in range(_D):
                j = s * _D + c
                m_copies.append(
                    pltpu.async_copy(
                        m_v.at[pl.ds(j * _NPW, _NPW)],
                        out_hbm.at[j, pl.ds(n0, _NPW)],
                        sem,
                    )
                )
            if s == 0:
                for p in range(3):
                    xr[p].wait()
                    xw[p] = x_write(p)
            elif s == 1:
                for p in range(3):
                    xw[p].wait()
                    xr[p + 4] = x_read(p + 4)
                xr[3].wait()
                xw[3] = x_write(3)
            else:
                xw[3].wait()
                xr[7] = x_read(7)
                for p in range(4, 6):
                    xr[p].wait()
                    xw[p] = x_write(p)

        xw[4].wait()
        xr[8] = x_read(8)
        for p in (6, 7, 8):
            xr[p].wait()
            xw[p] = x_write(p)
        for h in m_copies:
            h.wait()
        for p in (5, 6, 7, 8):
            xw[p].wait()

    return gather_kernel


_gather = _build_gather()


def kernel(X, kmer, emb_table):
    kmer_t = kmer.astype(jnp.int32).T.reshape(-1)
    m, xo = _gather(kmer_t, emb_table.reshape(-1), X.transpose(2, 1, 0))
    return (xo.transpose(2, 1, 0), m.T)


# speedup vs baseline: 1.2794x; 1.0036x over previous
import functools

import jax
import jax.numpy as jnp
from jax import lax
from jax.experimental import pallas as pl
from jax.experimental.pallas import tpu as pltpu
from jax.experimental.pallas import tpu_sc as plsc

_NC = 2
_NS = 16
_NW = _NC * _NS
_N = 16384
_K = 3
_D = 16
_J = _K * _D
_V = 1024
_NPW = _N // _NW
_L = 16
_GRP = _NPW // _L
_XP = 9
_XR = 20


def _build_gather():
    mesh = plsc.VectorSubcoreMesh(core_axis_name="c", subcore_axis_name="s")

    @functools.partial(
        pl.kernel,
        mesh=mesh,
        out_type=(
            jax.ShapeDtypeStruct((_J, _N), jnp.float32),
            jax.ShapeDtypeStruct((_XP, _XR, _N), jnp.float32),
        ),
        scratch_types=[
            pltpu.VMEM((_K * _NPW,), jnp.int32),
            pltpu.VMEM((_V * _D,), jnp.float32),
            pltpu.VMEM((_J * _NPW,), jnp.float32),
            pltpu.VMEM((4, _XR, _NPW), jnp.float32),
            pltpu.SemaphoreType.DMA,
            pltpu.SemaphoreType.DMA,
            pltpu.SemaphoreType.DMA,
        ],
        compiler_params=pltpu.CompilerParams(needs_layout_passes=False),
    )
    def gather_kernel(kmer_t_hbm, table_hbm, x_hbm, out_hbm, xout_hbm,
                      idx_v, tab_v, m_v, x_v, sem, xrsem, xwsem):
        wid = lax.axis_index("s") * _NC + lax.axis_index("c")
        n0 = wid * _NPW

        def x_read(p):
            return pltpu.async_copy(
                x_hbm.at[p, :, pl.ds(n0, _NPW)], x_v.at[p % 4], xrsem)

        def x_write(p):
            return pltpu.async_copy(
                x_v.at[p % 4], xout_hbm.at[p, :, pl.ds(n0, _NPW)], xwsem)

        xr = {p: x_read(p) for p in range(4)}
        xw = {}
        pltpu.sync_copy(table_hbm, tab_v)
        for k in range(_K):
            pltpu.sync_copy(
                kmer_t_hbm.at[pl.ds(k * _N + n0, _NPW)],
                idx_v.at[pl.ds(k * _NPW, _NPW)],
            )

        m_copies = []
        for s in range(_K):

            @plsc.parallel_loop(0, _GRP, unroll=1)
            def _body(i):
                base = i * _L
                voff = idx_v[pl.ds(s * _NPW + base, _L)] * _D
                vals = [plsc.load_gather(tab_v, [voff + c]) for c in range(_D)]
                for c in range(_D):
                    m_v[pl.ds((s * _D + c) * _NPW + base, _L)] = vals[c]

            for c in range(_D):
                j = s * _D + c
                m_copies.append(
                    pltpu.async_copy(
                        m_v.at[pl.ds(j * _NPW, _NPW)],
                        out_hbm.at[j, pl.ds(n0, _NPW)],
                        sem,
                    )
                )
            if s == 0:
                for p in range(3):
                    xr[p].wait()
                    xw[p] = x_write(p)
            elif s == 1:
                for p in range(3):
                    xw[p].wait()
                    xr[p + 4] = x_read(p + 4)
                xr[3].wait()
                xw[3] = x_write(3)
            else:
                xw[3].wait()
                xr[7] = x_read(7)
                for p in range(4, 6):
                    xr[p].wait()
                    xw[p] = x_write(p)

        xw[4].wait()
        xr[8] = x_read(8)
        for p in (6, 7, 8):
            xr[p].wait()
            xw[p] = x_write(p)
        for h in m_copies:
            h.wait()
        for p in (5, 6, 7, 8):
            xw[p].wait()

    return gather_kernel


_gather = _build_gather()


def kernel(X, kmer, emb_table):
    kmer_t = kmer.astype(jnp.int32).T.reshape(-1)
    m, xo = _gather(kmer_t, emb_table.reshape(-1), X.transpose(2, 1, 0))
    return (xo.transpose(2, 1, 0), m.T)


# speedup vs baseline: 1.3638x; 1.0660x over previous
import functools

import jax
import jax.numpy as jnp
from jax import lax
from jax.experimental import pallas as pl
from jax.experimental.pallas import tpu as pltpu
from jax.experimental.pallas import tpu_sc as plsc

_NC = 2
_NS = 16
_NW = _NC * _NS
_N = 16384
_K = 3
_D = 16
_J = _K * _D
_V = 1024
_NPW = _N // _NW
_L = 16
_GRP = _NPW // _L
_XP = 9
_XR = 20


def _build_gather():
    mesh = plsc.VectorSubcoreMesh(core_axis_name="c", subcore_axis_name="s")

    @functools.partial(
        pl.kernel,
        mesh=mesh,
        out_type=(
            jax.ShapeDtypeStruct((_J, _N), jnp.float32),
            jax.ShapeDtypeStruct((_XP, _XR, _N), jnp.float32),
        ),
        scratch_types=[
            pltpu.VMEM((_K * _NPW,), jnp.int32),
            pltpu.VMEM((_V * _D,), jnp.float32),
            pltpu.VMEM((_J * _NPW,), jnp.float32),
            pltpu.VMEM((5, _XR, _NPW), jnp.float32),
            pltpu.SemaphoreType.DMA,
            pltpu.SemaphoreType.DMA,
            pltpu.SemaphoreType.DMA,
            pltpu.SemaphoreType.DMA,
        ],
        compiler_params=pltpu.CompilerParams(needs_layout_passes=False),
    )
    def gather_kernel(kmer_t_hbm, table_hbm, x_hbm, out_hbm, xout_hbm,
                      idx_v, tab_v, m_v, x_v, sem, xrsem, xwsem, ssem):
        wid = lax.axis_index("s") * _NC + lax.axis_index("c")
        n0 = wid * _NPW

        def x_read(p):
            return pltpu.async_copy(
                x_hbm.at[p, :, pl.ds(n0, _NPW)], x_v.at[p % 5], xrsem)

        def x_write(p):
            return pltpu.async_copy(
                x_v.at[p % 5], xout_hbm.at[p, :, pl.ds(n0, _NPW)], xwsem)

        stage = [pltpu.async_copy(table_hbm, tab_v, ssem)]
        for k in range(_K):
            stage.append(
                pltpu.async_copy(
                    kmer_t_hbm.at[pl.ds(k * _N + n0, _NPW)],
                    idx_v.at[pl.ds(k * _NPW, _NPW)],
                    ssem,
                )
            )
        xr = {p: x_read(p) for p in range(5)}
        xw = {}
        for h in stage:
            h.wait()

        m_copies = []
        for s in range(_K):

            @plsc.parallel_loop(0, _GRP, unroll=1)
            def _body(i):
                base = i * _L
                voff = idx_v[pl.ds(s * _NPW + base, _L)] * _D
                vals = [plsc.load_gather(tab_v, [voff + c]) for c in range(_D)]
                for c in range(_D):
                    m_v[pl.ds((s * _D + c) * _NPW + base, _L)] = vals[c]

            for c in range(_D):
                j = s * _D + c
                m_copies.append(
                    pltpu.async_copy(
                        m_v.at[pl.ds(j * _NPW, _NPW)],
                        out_hbm.at[j, pl.ds(n0, _NPW)],
                        sem,
                    )
                )
            if s == 0:
                for p in range(3):
                    xr[p].wait()
                    xw[p] = x_write(p)
            elif s == 1:
                xw[0].wait()
                xr[5] = x_read(5)
                xw[1].wait()
                xr[6] = x_read(6)
                for p in (3, 4):
                    xr[p].wait()
                    xw[p] = x_write(p)
            else:
                xw[2].wait()
                xr[7] = x_read(7)
                xw[3].wait()
                xr[8] = x_read(8)
                for p in (5, 6):
                    xr[p].wait()
                    xw[p] = x_write(p)

        for p in (7, 8):
            xr[p].wait()
            xw[p] = x_write(p)
        for h in m_copies:
            h.wait()
        for p in (4, 5, 6, 7, 8):
            xw[p].wait()

    return gather_kernel


_gather = _build_gather()


def kernel(X, kmer, emb_table):
    kmer_t = kmer.astype(jnp.int32).T.reshape(-1)
    m, xo = _gather(kmer_t, emb_table.reshape(-1), X.transpose(2, 1, 0))
    return (xo.transpose(2, 1, 0), m.T)
